# baseline (speedup 1.0000x reference)
import functools
import jax
import jax.numpy as jnp
from jax import lax
from jax.experimental import pallas as pl
from jax.experimental.pallas import tpu as pltpu
from jax.experimental.pallas import tpu_sc as plsc

N_NODES = 10000
N_ROWS = 10016
PAD_DST = 10008
N_EDGES = 320000
D_IN = 128
D_HID = 128
N_CLASS = 40
D_PAD = 48

N_SC = 2
N_TILES = 16
N_WORKERS = N_SC * N_TILES
ROWS_PER_TILE = N_ROWS // N_TILES
ROW_BLK = 2504


def _edge_aggregate(sup, src2d, dst2d, d, chunk, slots, nbuf):
    mesh = plsc.VectorSubcoreMesh(core_axis_name="c", subcore_axis_name="s",
                                  num_cores=N_SC, num_subcores=N_TILES)

    def body(sup_hbm, src_hbm, dst_hbm, zeros_hbm, out_hbm,
             sidx_v, didx_v, rows_v, acc_sh, sem_g, sem_s):
        c = lax.axis_index("c")
        s = lax.axis_index("s")
        wid = c * N_TILES + s
        start = wid * slots

        pltpu.sync_copy(src_hbm.at[pl.ds(start, slots)], sidx_v)
        pltpu.sync_copy(dst_hbm.at[pl.ds(start, slots)], didx_v)
        pltpu.sync_copy(zeros_hbm,
                        acc_sh.at[pl.ds(s * ROWS_PER_TILE, ROWS_PER_TILE)])
        plsc.subcore_barrier()

        def fire_gather(t, b):
            pltpu.async_copy(sup_hbm.at[sidx_v.at[t]], rows_v.at[b],
                             sem_g.at[b])

        def wait_gather(t, b):
            pltpu.make_async_copy(sup_hbm.at[sidx_v.at[t]], rows_v.at[b],
                                  sem_g.at[b]).wait()

        def fire_scatter(t, b):
            pltpu.async_copy(rows_v.at[b], acc_sh.at[didx_v.at[t]],
                             sem_s.at[b], add=True)

        def wait_scatter(t, b):
            pltpu.make_async_copy(rows_v.at[b], acc_sh.at[didx_v.at[t]],
                                  sem_s.at[b]).wait()

        for b in range(nbuf - 1):
            fire_gather(b, b)

        def slot_step(t, sb, first=False, last_grp=False):
            sbp = (sb + nbuf - 1) % nbuf
            wait_gather(t, sb)
            fire_scatter(t, sb)
            if not first:
                wait_scatter(t - 1, sbp)
            if not last_grp:
                fire_gather(t + nbuf - 1, sbp)

        for sb in range(nbuf):
            slot_step(sb, sb, first=(sb == 0))

        def group(g, carry):
            t0 = g * nbuf
            for sb in range(nbuf):
                slot_step(t0 + sb, sb)
            return carry

        lax.fori_loop(1, slots // nbuf - 1, group, 0)

        t0 = slots - nbuf
        for sb in range(nbuf):
            slot_step(t0 + sb, sb, last_grp=(sb != 0))
        wait_scatter(slots - 1, (slots - 1) % nbuf)

        plsc.subcore_barrier()
        pltpu.sync_copy(acc_sh.at[pl.ds(s * ROWS_PER_TILE, ROWS_PER_TILE)],
                        out_hbm.at[c, s])

    kern = pl.kernel(
        body,
        out_type=jax.ShapeDtypeStruct((N_SC, N_TILES, ROWS_PER_TILE, d),
                                      jnp.float32),
        mesh=mesh,
        scratch_types=[
            pltpu.VMEM((slots, chunk), jnp.int32),
            pltpu.VMEM((slots, chunk), jnp.int32),
            pltpu.VMEM((nbuf, chunk, d), jnp.float32),
            pltpu.VMEM_SHARED((N_ROWS, d), jnp.float32),
            pltpu.SemaphoreType.DMA((nbuf,)),
            pltpu.SemaphoreType.DMA((nbuf,)),
        ],
        compiler_params=pltpu.CompilerParams(use_tc_tiling_on_sc=False),
    )
    zeros = jnp.zeros((ROWS_PER_TILE, d), jnp.float32)
    out = kern(sup, src2d, dst2d, zeros)
    return out.reshape(N_SC, N_ROWS, d)


def _pad_edges(src, dst, chunk, slots):
    n_pad = N_WORKERS * slots * chunk - N_EDGES
    src2d = jnp.concatenate(
        [src, jnp.zeros((n_pad,), jnp.int32)]).reshape(-1, chunk)
    pad_dst = N_NODES + (jnp.arange(n_pad, dtype=jnp.int32) % (N_ROWS - N_NODES))
    dst2d = jnp.concatenate([dst, pad_dst]).reshape(-1, chunk)
    return src2d, dst2d


def _matmul1(x, w):
    def body(x_ref, w_ref, o_ref):
        o_ref[...] = jnp.dot(x_ref[...], w_ref[...],
                             preferred_element_type=jnp.float32)

    return pl.pallas_call(
        body,
        grid=(4,),
        in_specs=[
            pl.BlockSpec((ROW_BLK, D_IN), lambda i: (i, 0)),
            pl.BlockSpec((D_IN, D_HID), lambda i: (0, 0)),
        ],
        out_specs=pl.BlockSpec((ROW_BLK, D_HID), lambda i: (i, 0)),
        out_shape=jax.ShapeDtypeStruct((N_ROWS, D_HID), jnp.float32),
    )(x, w)


def _mid(p, b1, w2p):
    def body(p_ref, b_ref, w_ref, o_ref):
        h = jnp.maximum(p_ref[0] + p_ref[1] + b_ref[...], 0.0)
        o_ref[...] = jnp.dot(h, w_ref[...], preferred_element_type=jnp.float32)

    return pl.pallas_call(
        body,
        grid=(4,),
        in_specs=[
            pl.BlockSpec((N_SC, ROW_BLK, D_HID), lambda i: (0, i, 0)),
            pl.BlockSpec((1, D_HID), lambda i: (0, 0)),
            pl.BlockSpec((D_HID, D_PAD), lambda i: (0, 0)),
        ],
        out_specs=pl.BlockSpec((ROW_BLK, D_PAD), lambda i: (i, 0)),
        out_shape=jax.ShapeDtypeStruct((N_ROWS, D_PAD), jnp.float32),
    )(p, b1, w2p)


def _final(q, b2p):
    def body(q_ref, b_ref, o_ref):
        o_ref[...] = q_ref[0] + q_ref[1] + b_ref[...]

    return pl.pallas_call(
        body,
        grid=(4,),
        in_specs=[
            pl.BlockSpec((N_SC, ROW_BLK, D_PAD), lambda i: (0, i, 0)),
            pl.BlockSpec((1, D_PAD), lambda i: (0, 0)),
        ],
        out_specs=pl.BlockSpec((ROW_BLK, D_PAD), lambda i: (i, 0)),
        out_shape=jax.ShapeDtypeStruct((N_ROWS, D_PAD), jnp.float32),
    )(q, b2p)


@jax.jit
def kernel(x, adj, W1, b1, W2, b2):
    src1, dst1 = _pad_edges(adj[0], adj[1], 112, 90)
    src2, dst2 = _pad_edges(adj[0], adj[1], 128, 80)
    xp = jnp.pad(x, ((0, N_ROWS - N_NODES), (0, 0)))
    w2p = jnp.pad(W2, ((0, 0), (0, D_PAD - N_CLASS)))
    b2p = jnp.pad(b2, (0, D_PAD - N_CLASS)).reshape(1, D_PAD)
    b1r = b1.reshape(1, D_HID)

    support1 = _matmul1(xp, W1)
    p1 = _edge_aggregate(support1, src1, dst1, D_HID, 112, 90, 2)
    support2 = _mid(p1, b1r, w2p)
    q = _edge_aggregate(support2, src2, dst2, D_PAD, 128, 80, 4)
    out = _final(q, b2p)
    return out[:N_NODES, :N_CLASS]


# speedup vs baseline: 1.4571x; 1.4571x over previous
import functools
import jax
import jax.numpy as jnp
from jax import lax
from jax.experimental import pallas as pl
from jax.experimental.pallas import tpu as pltpu
from jax.experimental.pallas import tpu_sc as plsc

N_NODES = 10000
N_ROWS = 10016
PAD_DST = 10008
N_EDGES = 320000
D_IN = 128
D_HID = 128
N_CLASS = 40
D_PAD = 48

N_SC = 2
N_TILES = 16
N_WORKERS = N_SC * N_TILES
ROWS_PER_TILE = N_ROWS // N_TILES
ROW_BLK = 2504
CHUNK = 88
N_CHUNKS = 3648


def _edge_aggregate(sup, src2d, dst2d, d, chunk, s0, s1, nbuf):
    assert s0 % nbuf == 0 and s1 % nbuf == 0 and s0 >= s1 >= nbuf

    mesh = plsc.VectorSubcoreMesh(core_axis_name="c", subcore_axis_name="s",
                                  num_cores=N_SC, num_subcores=N_TILES)

    def body(sup_hbm, src_hbm, dst_hbm, zeros_hbm, out_hbm,
             sidx_v, didx_v, rows_v, acc_sh, sem_g, sem_s):
        c = lax.axis_index("c")
        s = lax.axis_index("s")
        nt = jnp.where(c == 0, s0, s1)

        @pl.when(c == 0)
        def _():
            start = s * s0
            pltpu.sync_copy(src_hbm.at[pl.ds(start, s0)], sidx_v)
            pltpu.sync_copy(dst_hbm.at[pl.ds(start, s0)], didx_v)

        @pl.when(c == 1)
        def _():
            start = N_TILES * s0 + s * s1
            pltpu.sync_copy(src_hbm.at[pl.ds(start, s1)],
                            sidx_v.at[pl.ds(0, s1)])
            pltpu.sync_copy(dst_hbm.at[pl.ds(start, s1)],
                            didx_v.at[pl.ds(0, s1)])

        pltpu.sync_copy(zeros_hbm,
                        acc_sh.at[pl.ds(s * ROWS_PER_TILE, ROWS_PER_TILE)])
        plsc.subcore_barrier()

        def fire_gather(t, b):
            pltpu.async_copy(sup_hbm.at[sidx_v.at[t]], rows_v.at[b],
                             sem_g.at[b])

        def wait_gather(t, b):
            pltpu.make_async_copy(sup_hbm.at[sidx_v.at[t]], rows_v.at[b],
                                  sem_g.at[b]).wait()

        def fire_scatter(t, b):
            pltpu.async_copy(rows_v.at[b], acc_sh.at[didx_v.at[t]],
                             sem_s.at[b], add=True)

        def wait_scatter(t, b):
            pltpu.make_async_copy(rows_v.at[b], acc_sh.at[didx_v.at[t]],
                                  sem_s.at[b]).wait()

        for b in range(nbuf - 1):
            fire_gather(b, b)

        def slot_step(t, sb, first=False, last_grp=False):
            sbp = (sb + nbuf - 1) % nbuf
            wait_gather(t, sb)
            fire_scatter(t, sb)
            if not first:
                wait_scatter(t - 1, sbp)
            if not last_grp:
                fire_gather(t + nbuf - 1, sbp)

        for sb in range(nbuf):
            slot_step(sb, sb, first=(sb == 0))

        def group(g, carry):
            t0 = g * nbuf
            for sb in range(nbuf):
                slot_step(t0 + sb, sb)
            return carry

        lax.fori_loop(1, nt // nbuf - 1, group, 0)

        t0 = nt - nbuf
        for sb in range(nbuf):
            slot_step(t0 + sb, sb, last_grp=(sb != 0))
        wait_scatter(nt - 1, nbuf - 1)

        plsc.subcore_barrier()
        pltpu.sync_copy(acc_sh.at[pl.ds(s * ROWS_PER_TILE, ROWS_PER_TILE)],
                        out_hbm.at[c, s])

    kern = pl.kernel(
        body,
        out_type=jax.ShapeDtypeStruct((N_SC, N_TILES, ROWS_PER_TILE, d),
                                      jnp.float32),
        mesh=mesh,
        scratch_types=[
            pltpu.VMEM((s0, chunk), jnp.int32),
            pltpu.VMEM((s0, chunk), jnp.int32),
            pltpu.VMEM((nbuf, chunk, d), jnp.float32),
            pltpu.VMEM_SHARED((N_ROWS, d), jnp.float32),
            pltpu.SemaphoreType.DMA((nbuf,)),
            pltpu.SemaphoreType.DMA((nbuf,)),
        ],
        compiler_params=pltpu.CompilerParams(use_tc_tiling_on_sc=False),
    )
    zeros = jnp.zeros((ROWS_PER_TILE, d), jnp.float32)
    out = kern(sup, src2d, dst2d, zeros)
    return out.reshape(N_SC, N_ROWS, d)


def _pad_edges(src, dst, chunk, n_chunks):
    n_pad = n_chunks * chunk - N_EDGES
    src2d = jnp.concatenate(
        [src, jnp.zeros((n_pad,), jnp.int32)]).reshape(-1, chunk)
    pad_dst = N_NODES + (jnp.arange(n_pad, dtype=jnp.int32) % (N_ROWS - N_NODES))
    dst2d = jnp.concatenate([dst, pad_dst]).reshape(-1, chunk)
    return src2d, dst2d


def _matmul1(x, w):
    def body(x_ref, w_ref, o_ref):
        o_ref[...] = jnp.dot(x_ref[...], w_ref[...],
                             preferred_element_type=jnp.float32)

    return pl.pallas_call(
        body,
        grid=(4,),
        in_specs=[
            pl.BlockSpec((ROW_BLK, D_IN), lambda i: (i, 0)),
            pl.BlockSpec((D_IN, D_HID), lambda i: (0, 0)),
        ],
        out_specs=pl.BlockSpec((ROW_BLK, D_HID), lambda i: (i, 0)),
        out_shape=jax.ShapeDtypeStruct((N_ROWS, D_HID), jnp.float32),
    )(x, w)


def _mid(p, b1, w2p):
    def body(p_ref, b_ref, w_ref, o_ref):
        h = jnp.maximum(p_ref[0] + p_ref[1] + b_ref[...], 0.0)
        o_ref[...] = jnp.dot(h, w_ref[...], preferred_element_type=jnp.float32)

    return pl.pallas_call(
        body,
        grid=(4,),
        in_specs=[
            pl.BlockSpec((N_SC, ROW_BLK, D_HID), lambda i: (0, i, 0)),
            pl.BlockSpec((1, D_HID), lambda i: (0, 0)),
            pl.BlockSpec((D_HID, D_PAD), lambda i: (0, 0)),
        ],
        out_specs=pl.BlockSpec((ROW_BLK, D_PAD), lambda i: (i, 0)),
        out_shape=jax.ShapeDtypeStruct((N_ROWS, D_PAD), jnp.float32),
    )(p, b1, w2p)


def _final(q, b2p):
    def body(q_ref, b_ref, o_ref):
        o_ref[...] = q_ref[0] + q_ref[1] + b_ref[...]

    return pl.pallas_call(
        body,
        grid=(4,),
        in_specs=[
            pl.BlockSpec((N_SC, ROW_BLK, D_PAD), lambda i: (0, i, 0)),
            pl.BlockSpec((1, D_PAD), lambda i: (0, 0)),
        ],
        out_specs=pl.BlockSpec((ROW_BLK, D_PAD), lambda i: (i, 0)),
        out_shape=jax.ShapeDtypeStruct((N_ROWS, D_PAD), jnp.float32),
    )(q, b2p)


@jax.jit
def kernel(x, adj, W1, b1, W2, b2):
    src2d, dst2d = _pad_edges(adj[0], adj[1], CHUNK, N_CHUNKS)
    xp = jnp.pad(x, ((0, N_ROWS - N_NODES), (0, 0)))
    w2p = jnp.pad(W2, ((0, 0), (0, D_PAD - N_CLASS)))
    b2p = jnp.pad(b2, (0, D_PAD - N_CLASS)).reshape(1, D_PAD)
    b1r = b1.reshape(1, D_HID)

    support1 = _matmul1(xp, W1)
    p1 = _edge_aggregate(support1, src2d, dst2d, D_HID, CHUNK, 152, 76, 2)
    support2 = _mid(p1, b1r, w2p)
    q = _edge_aggregate(support2, src2d, dst2d, D_PAD, CHUNK, 168, 60, 4)
    out = _final(q, b2p)
    return out[:N_NODES, :N_CLASS]


# speedup vs baseline: 1.9758x; 1.3560x over previous
import functools
import jax
import jax.numpy as jnp
from jax import lax
from jax.experimental import pallas as pl
from jax.experimental.pallas import tpu as pltpu
from jax.experimental.pallas import tpu_sc as plsc

N_NODES = 10000
N_ROWS = 10016
PAD_DST = 10008
N_EDGES = 320000
D_IN = 128
D_HID = 128
N_CLASS = 40
D_PAD = 48

N_SC = 2
N_TILES = 16
N_WORKERS = N_SC * N_TILES
ROWS_PER_TILE = N_ROWS // N_TILES
ROW_BLK = 2504
CHUNK = 88
N_CHUNKS = 3648


def _edge_aggregate(sup, src2d, dst2d, d, chunk, s0, s1, nbuf, dtype):
    assert s0 % nbuf == 0 and s1 % nbuf == 0 and s0 >= s1 >= nbuf

    mesh = plsc.VectorSubcoreMesh(core_axis_name="c", subcore_axis_name="s",
                                  num_cores=N_SC, num_subcores=N_TILES)

    def body(sup_hbm, src_hbm, dst_hbm, zeros_hbm, out_hbm,
             sidx_v, didx_v, rows_v, acc_sh, sem_g, sem_s):
        c = lax.axis_index("c")
        s = lax.axis_index("s")
        nt = jnp.where(c == 0, s0, s1)

        @pl.when(c == 0)
        def _():
            start = s * s0
            pltpu.sync_copy(src_hbm.at[pl.ds(start, s0)], sidx_v)
            pltpu.sync_copy(dst_hbm.at[pl.ds(start, s0)], didx_v)

        @pl.when(c == 1)
        def _():
            start = N_TILES * s0 + s * s1
            pltpu.sync_copy(src_hbm.at[pl.ds(start, s1)],
                            sidx_v.at[pl.ds(0, s1)])
            pltpu.sync_copy(dst_hbm.at[pl.ds(start, s1)],
                            didx_v.at[pl.ds(0, s1)])

        pltpu.sync_copy(zeros_hbm,
                        acc_sh.at[pl.ds(s * ROWS_PER_TILE, ROWS_PER_TILE)])
        plsc.subcore_barrier()

        def fire_gather(t, b):
            pltpu.async_copy(sup_hbm.at[sidx_v.at[t]], rows_v.at[b],
                             sem_g.at[b])

        def wait_gather(t, b):
            pltpu.make_async_copy(sup_hbm.at[sidx_v.at[t]], rows_v.at[b],
                                  sem_g.at[b]).wait()

        def fire_scatter(t, b):
            pltpu.async_copy(rows_v.at[b], acc_sh.at[didx_v.at[t]],
                             sem_s.at[b], add=True)

        def wait_scatter(t, b):
            pltpu.make_async_copy(rows_v.at[b], acc_sh.at[didx_v.at[t]],
                                  sem_s.at[b]).wait()

        for b in range(nbuf - 1):
            fire_gather(b, b)

        def slot_step(t, sb, first=False, last_grp=False):
            sbp = (sb + nbuf - 1) % nbuf
            wait_gather(t, sb)
            fire_scatter(t, sb)
            if not first:
                wait_scatter(t - 1, sbp)
            if not last_grp:
                fire_gather(t + nbuf - 1, sbp)

        for sb in range(nbuf):
            slot_step(sb, sb, first=(sb == 0))

        def group(g, carry):
            t0 = g * nbuf
            for sb in range(nbuf):
                slot_step(t0 + sb, sb)
            return carry

        lax.fori_loop(1, nt // nbuf - 1, group, 0)

        t0 = nt - nbuf
        for sb in range(nbuf):
            slot_step(t0 + sb, sb, last_grp=(sb != 0))
        wait_scatter(nt - 1, nbuf - 1)

        plsc.subcore_barrier()
        pltpu.sync_copy(acc_sh.at[pl.ds(s * ROWS_PER_TILE, ROWS_PER_TILE)],
                        out_hbm.at[c, s])

    kern = pl.kernel(
        body,
        out_type=jax.ShapeDtypeStruct((N_SC, N_TILES, ROWS_PER_TILE, d),
                                      dtype),
        mesh=mesh,
        scratch_types=[
            pltpu.VMEM((s0, chunk), jnp.int32),
            pltpu.VMEM((s0, chunk), jnp.int32),
            pltpu.VMEM((nbuf, chunk, d), dtype),
            pltpu.VMEM_SHARED((N_ROWS, d), dtype),
            pltpu.SemaphoreType.DMA((nbuf,)),
            pltpu.SemaphoreType.DMA((nbuf,)),
        ],
        compiler_params=pltpu.CompilerParams(use_tc_tiling_on_sc=False),
    )
    zeros = jnp.zeros((ROWS_PER_TILE, d), dtype)
    out = kern(sup, src2d, dst2d, zeros)
    return out.reshape(N_SC, N_ROWS, d)


def _pad_edges(src, dst, chunk, n_chunks):
    n_pad = n_chunks * chunk - N_EDGES
    src2d = jnp.concatenate(
        [src, jnp.zeros((n_pad,), jnp.int32)]).reshape(-1, chunk)
    pad_dst = N_NODES + (jnp.arange(n_pad, dtype=jnp.int32) % (N_ROWS - N_NODES))
    dst2d = jnp.concatenate([dst, pad_dst]).reshape(-1, chunk)
    return src2d, dst2d


def _matmul1(x, w):
    def body(x_ref, w_ref, o_ref):
        o_ref[...] = jnp.dot(x_ref[...], w_ref[...],
                             preferred_element_type=jnp.float32
                             ).astype(jnp.bfloat16)

    return pl.pallas_call(
        body,
        grid=(4,),
        in_specs=[
            pl.BlockSpec((ROW_BLK, D_IN), lambda i: (i, 0)),
            pl.BlockSpec((D_IN, D_HID), lambda i: (0, 0)),
        ],
        out_specs=pl.BlockSpec((ROW_BLK, D_HID), lambda i: (i, 0)),
        out_shape=jax.ShapeDtypeStruct((N_ROWS, D_HID), jnp.bfloat16),
    )(x, w)


def _mid(p, b1, w2p):
    def body(p_ref, b_ref, w_ref, o_ref):
        psum = p_ref[0].astype(jnp.float32) + p_ref[1].astype(jnp.float32)
        h = jnp.maximum(psum + b_ref[...], 0.0)
        o_ref[...] = jnp.dot(h, w_ref[...], preferred_element_type=jnp.float32)

    return pl.pallas_call(
        body,
        grid=(4,),
        in_specs=[
            pl.BlockSpec((N_SC, ROW_BLK, D_HID), lambda i: (0, i, 0)),
            pl.BlockSpec((1, D_HID), lambda i: (0, 0)),
            pl.BlockSpec((D_HID, D_PAD), lambda i: (0, 0)),
        ],
        out_specs=pl.BlockSpec((ROW_BLK, D_PAD), lambda i: (i, 0)),
        out_shape=jax.ShapeDtypeStruct((N_ROWS, D_PAD), jnp.float32),
    )(p, b1, w2p)


def _final(q, b2p):
    def body(q_ref, b_ref, o_ref):
        o_ref[...] = q_ref[0] + q_ref[1] + b_ref[...]

    return pl.pallas_call(
        body,
        grid=(4,),
        in_specs=[
            pl.BlockSpec((N_SC, ROW_BLK, D_PAD), lambda i: (0, i, 0)),
            pl.BlockSpec((1, D_PAD), lambda i: (0, 0)),
        ],
        out_specs=pl.BlockSpec((ROW_BLK, D_PAD), lambda i: (i, 0)),
        out_shape=jax.ShapeDtypeStruct((N_ROWS, D_PAD), jnp.float32),
    )(q, b2p)


@jax.jit
def kernel(x, adj, W1, b1, W2, b2):
    src2d, dst2d = _pad_edges(adj[0], adj[1], CHUNK, N_CHUNKS)
    xp = jnp.pad(x, ((0, N_ROWS - N_NODES), (0, 0)))
    w2p = jnp.pad(W2, ((0, 0), (0, D_PAD - N_CLASS)))
    b2p = jnp.pad(b2, (0, D_PAD - N_CLASS)).reshape(1, D_PAD)
    b1r = b1.reshape(1, D_HID)

    support1 = _matmul1(xp, W1)
    p1 = _edge_aggregate(support1, src2d, dst2d, D_HID, CHUNK, 152, 76, 4,
                         jnp.bfloat16)
    support2 = _mid(p1, b1r, w2p)
    q = _edge_aggregate(support2, src2d, dst2d, D_PAD, CHUNK, 168, 60, 4,
                        jnp.float32)
    out = _final(q, b2p)
    return out[:N_NODES, :N_CLASS]


# speedup vs baseline: 2.0691x; 1.0472x over previous
import functools
import jax
import jax.numpy as jnp
from jax import lax
from jax.experimental import pallas as pl
from jax.experimental.pallas import tpu as pltpu
from jax.experimental.pallas import tpu_sc as plsc

N_NODES = 10000
N_ROWS = 10016
PAD_DST = 10008
N_EDGES = 320000
D_IN = 128
D_HID = 128
N_CLASS = 40
D_PAD = 64

N_SC = 2
N_TILES = 16
N_WORKERS = N_SC * N_TILES
ROWS_PER_TILE = N_ROWS // N_TILES
ROW_BLK = 2504
CHUNK = 88
N_CHUNKS = 3648


def _edge_aggregate(sup, src2d, dst2d, d, chunk, s0, s1, nbuf, dtype):
    assert s0 % nbuf == 0 and s1 % nbuf == 0 and s0 >= s1 >= nbuf

    mesh = plsc.VectorSubcoreMesh(core_axis_name="c", subcore_axis_name="s",
                                  num_cores=N_SC, num_subcores=N_TILES)

    def body(sup_hbm, src_hbm, dst_hbm, zeros_hbm, out_hbm,
             sidx_v, didx_v, rows_v, acc_sh, sem_g, sem_s):
        c = lax.axis_index("c")
        s = lax.axis_index("s")
        nt = jnp.where(c == 0, s0, s1)

        @pl.when(c == 0)
        def _():
            start = s * s0
            pltpu.sync_copy(src_hbm.at[pl.ds(start, s0)], sidx_v)
            pltpu.sync_copy(dst_hbm.at[pl.ds(start, s0)], didx_v)

        @pl.when(c == 1)
        def _():
            start = N_TILES * s0 + s * s1
            pltpu.sync_copy(src_hbm.at[pl.ds(start, s1)],
                            sidx_v.at[pl.ds(0, s1)])
            pltpu.sync_copy(dst_hbm.at[pl.ds(start, s1)],
                            didx_v.at[pl.ds(0, s1)])

        pltpu.sync_copy(zeros_hbm,
                        acc_sh.at[pl.ds(s * ROWS_PER_TILE, ROWS_PER_TILE)])
        plsc.subcore_barrier()

        def fire_gather(t, b):
            pltpu.async_copy(sup_hbm.at[sidx_v.at[t]], rows_v.at[b],
                             sem_g.at[b])

        def wait_gather(t, b):
            pltpu.make_async_copy(sup_hbm.at[sidx_v.at[t]], rows_v.at[b],
                                  sem_g.at[b]).wait()

        def fire_scatter(t, b):
            pltpu.async_copy(rows_v.at[b], acc_sh.at[didx_v.at[t]],
                             sem_s.at[b], add=True)

        def wait_scatter(t, b):
            pltpu.make_async_copy(rows_v.at[b], acc_sh.at[didx_v.at[t]],
                                  sem_s.at[b]).wait()

        for b in range(nbuf - 1):
            fire_gather(b, b)

        def slot_step(t, sb, first=False, last_grp=False):
            sbp = (sb + nbuf - 1) % nbuf
            wait_gather(t, sb)
            fire_scatter(t, sb)
            if not first:
                wait_scatter(t - 1, sbp)
            if not last_grp:
                fire_gather(t + nbuf - 1, sbp)

        for sb in range(nbuf):
            slot_step(sb, sb, first=(sb == 0))

        def group(g, carry):
            t0 = g * nbuf
            for sb in range(nbuf):
                slot_step(t0 + sb, sb)
            return carry

        lax.fori_loop(1, nt // nbuf - 1, group, 0)

        t0 = nt - nbuf
        for sb in range(nbuf):
            slot_step(t0 + sb, sb, last_grp=(sb != 0))
        wait_scatter(nt - 1, nbuf - 1)

        plsc.subcore_barrier()
        pltpu.sync_copy(acc_sh.at[pl.ds(s * ROWS_PER_TILE, ROWS_PER_TILE)],
                        out_hbm.at[c, s])

    kern = pl.kernel(
        body,
        out_type=jax.ShapeDtypeStruct((N_SC, N_TILES, ROWS_PER_TILE, d),
                                      dtype),
        mesh=mesh,
        scratch_types=[
            pltpu.VMEM((s0, chunk), jnp.int32),
            pltpu.VMEM((s0, chunk), jnp.int32),
            pltpu.VMEM((nbuf, chunk, d), dtype),
            pltpu.VMEM_SHARED((N_ROWS, d), dtype),
            pltpu.SemaphoreType.DMA((nbuf,)),
            pltpu.SemaphoreType.DMA((nbuf,)),
        ],
        compiler_params=pltpu.CompilerParams(use_tc_tiling_on_sc=False),
    )
    zeros = jnp.zeros((ROWS_PER_TILE, d), dtype)
    out = kern(sup, src2d, dst2d, zeros)
    return out.reshape(N_SC, N_ROWS, d)


def _pad_edges(src, dst, chunk, n_chunks):
    n_pad = n_chunks * chunk - N_EDGES
    src2d = jnp.concatenate(
        [src, jnp.zeros((n_pad,), jnp.int32)]).reshape(-1, chunk)
    pad_dst = N_NODES + (jnp.arange(n_pad, dtype=jnp.int32) % (N_ROWS - N_NODES))
    dst2d = jnp.concatenate([dst, pad_dst]).reshape(-1, chunk)
    return src2d, dst2d


def _matmul1(x, w):
    def body(x_ref, w_ref, o_ref):
        o_ref[...] = jnp.dot(x_ref[...], w_ref[...],
                             preferred_element_type=jnp.float32
                             ).astype(jnp.bfloat16)

    return pl.pallas_call(
        body,
        grid=(4,),
        in_specs=[
            pl.BlockSpec((ROW_BLK, D_IN), lambda i: (i, 0)),
            pl.BlockSpec((D_IN, D_HID), lambda i: (0, 0)),
        ],
        out_specs=pl.BlockSpec((ROW_BLK, D_HID), lambda i: (i, 0)),
        out_shape=jax.ShapeDtypeStruct((N_ROWS, D_HID), jnp.bfloat16),
    )(x, w)


def _mid(p, b1, w2p):
    def body(p_ref, b_ref, w_ref, o_ref):
        psum = p_ref[0].astype(jnp.float32) + p_ref[1].astype(jnp.float32)
        h = jnp.maximum(psum + b_ref[...], 0.0)
        o_ref[...] = jnp.dot(h, w_ref[...], preferred_element_type=jnp.float32
                             ).astype(jnp.bfloat16)

    return pl.pallas_call(
        body,
        grid=(4,),
        in_specs=[
            pl.BlockSpec((N_SC, ROW_BLK, D_HID), lambda i: (0, i, 0)),
            pl.BlockSpec((1, D_HID), lambda i: (0, 0)),
            pl.BlockSpec((D_HID, D_PAD), lambda i: (0, 0)),
        ],
        out_specs=pl.BlockSpec((ROW_BLK, D_PAD), lambda i: (i, 0)),
        out_shape=jax.ShapeDtypeStruct((N_ROWS, D_PAD), jnp.bfloat16),
    )(p, b1, w2p)


def _final(q, b2p):
    def body(q_ref, b_ref, o_ref):
        o_ref[...] = (q_ref[0].astype(jnp.float32)
                      + q_ref[1].astype(jnp.float32) + b_ref[...])

    return pl.pallas_call(
        body,
        grid=(4,),
        in_specs=[
            pl.BlockSpec((N_SC, ROW_BLK, D_PAD), lambda i: (0, i, 0)),
            pl.BlockSpec((1, D_PAD), lambda i: (0, 0)),
        ],
        out_specs=pl.BlockSpec((ROW_BLK, D_PAD), lambda i: (i, 0)),
        out_shape=jax.ShapeDtypeStruct((N_ROWS, D_PAD), jnp.float32),
    )(q, b2p)


@jax.jit
def kernel(x, adj, W1, b1, W2, b2):
    src2d, dst2d = _pad_edges(adj[0], adj[1], CHUNK, N_CHUNKS)
    xp = jnp.pad(x, ((0, N_ROWS - N_NODES), (0, 0)))
    w2p = jnp.pad(W2, ((0, 0), (0, D_PAD - N_CLASS)))
    b2p = jnp.pad(b2, (0, D_PAD - N_CLASS)).reshape(1, D_PAD)
    b1r = b1.reshape(1, D_HID)

    support1 = _matmul1(xp, W1)
    p1 = _edge_aggregate(support1, src2d, dst2d, D_HID, CHUNK, 152, 76, 4,
                         jnp.bfloat16)
    support2 = _mid(p1, b1r, w2p)
    q = _edge_aggregate(support2, src2d, dst2d, D_PAD, CHUNK, 168, 60, 4,
                        jnp.bfloat16)
    out = _final(q, b2p)
    return out[:N_NODES, :N_CLASS]


# speedup vs baseline: 2.2336x; 1.0795x over previous
import functools
import jax
import jax.numpy as jnp
from jax import lax
from jax.experimental import pallas as pl
from jax.experimental.pallas import tpu as pltpu
from jax.experimental.pallas import tpu_sc as plsc

N_NODES = 10000
N_EDGES = 320000
D_IN = 128
D_HID = 128
N_CLASS = 40
D_PAD = 64

N_SC = 2
N_TILES = 16
ROWS_PER_TILE = N_NODES // N_TILES
ROW_BLK = 2000
CHUNK = 100
S0 = 124
S1 = 76
NBUF = 4


def _edge_aggregate(sup, src2d, dst2d, d, dtype):
    mesh = plsc.VectorSubcoreMesh(core_axis_name="c", subcore_axis_name="s",
                                  num_cores=N_SC, num_subcores=N_TILES)

    def body(sup_hbm, src_hbm, dst_hbm, zeros_hbm, out_hbm,
             sidx_v, didx_v, rows_v, acc_sh, sem_g, sem_s):
        c = lax.axis_index("c")
        s = lax.axis_index("s")
        nt = jnp.where(c == 0, S0, S1)

        @pl.when(c == 0)
        def _():
            start = s * S0
            pltpu.sync_copy(src_hbm.at[pl.ds(start, S0)], sidx_v)
            pltpu.sync_copy(dst_hbm.at[pl.ds(start, S0)], didx_v)

        @pl.when(c == 1)
        def _():
            start = N_TILES * S0 + s * S1
            pltpu.sync_copy(src_hbm.at[pl.ds(start, S1)],
                            sidx_v.at[pl.ds(0, S1)])
            pltpu.sync_copy(dst_hbm.at[pl.ds(start, S1)],
                            didx_v.at[pl.ds(0, S1)])

        pltpu.sync_copy(zeros_hbm,
                        acc_sh.at[pl.ds(s * ROWS_PER_TILE, ROWS_PER_TILE)])
        plsc.subcore_barrier()

        def fire_gather(t, b):
            pltpu.async_copy(sup_hbm.at[sidx_v.at[t]], rows_v.at[b],
                             sem_g.at[b])

        def wait_gather(t, b):
            pltpu.make_async_copy(sup_hbm.at[sidx_v.at[t]], rows_v.at[b],
                                  sem_g.at[b]).wait()

        def fire_scatter(t, b):
            pltpu.async_copy(rows_v.at[b], acc_sh.at[didx_v.at[t]],
                             sem_s.at[b], add=True)

        def wait_scatter(t, b):
            pltpu.make_async_copy(rows_v.at[b], acc_sh.at[didx_v.at[t]],
                                  sem_s.at[b]).wait()

        for b in range(NBUF - 1):
            fire_gather(b, b)

        def slot_step(t, sb, first=False, last_grp=False):
            sbp = (sb + NBUF - 1) % NBUF
            wait_gather(t, sb)
            fire_scatter(t, sb)
            if not first:
                wait_scatter(t - 1, sbp)
            if not last_grp:
                fire_gather(t + NBUF - 1, sbp)

        for sb in range(NBUF):
            slot_step(sb, sb, first=(sb == 0))

        def group(g, carry):
            t0 = g * NBUF
            for sb in range(NBUF):
                slot_step(t0 + sb, sb)
            return carry

        lax.fori_loop(1, nt // NBUF - 1, group, 0)

        t0 = nt - NBUF
        for sb in range(NBUF):
            slot_step(t0 + sb, sb, last_grp=(sb != 0))
        wait_scatter(nt - 1, NBUF - 1)

        plsc.subcore_barrier()
        pltpu.sync_copy(acc_sh.at[pl.ds(s * ROWS_PER_TILE, ROWS_PER_TILE)],
                        out_hbm.at[c, pl.ds(s * ROWS_PER_TILE,
                                            ROWS_PER_TILE)])

    kern = pl.kernel(
        body,
        out_type=jax.ShapeDtypeStruct((N_SC, N_NODES, d), dtype),
        mesh=mesh,
        scratch_types=[
            pltpu.VMEM((S0, CHUNK), jnp.int32),
            pltpu.VMEM((S0, CHUNK), jnp.int32),
            pltpu.VMEM((NBUF, CHUNK, d), dtype),
            pltpu.VMEM_SHARED((N_NODES, d), dtype),
            pltpu.SemaphoreType.DMA((NBUF,)),
            pltpu.SemaphoreType.DMA((NBUF,)),
        ],
        compiler_params=pltpu.CompilerParams(use_tc_tiling_on_sc=False),
    )
    zeros = jnp.zeros((ROWS_PER_TILE, d), dtype)
    return kern(sup, src2d, dst2d, zeros)


def _matmul1(x, w):
    def body(x_ref, w_ref, o_ref):
        o_ref[...] = jnp.dot(x_ref[...], w_ref[...],
                             preferred_element_type=jnp.float32
                             ).astype(jnp.bfloat16)

    return pl.pallas_call(
        body,
        grid=(5,),
        in_specs=[
            pl.BlockSpec((ROW_BLK, D_IN), lambda i: (i, 0)),
            pl.BlockSpec((D_IN, D_HID), lambda i: (0, 0)),
        ],
        out_specs=pl.BlockSpec((ROW_BLK, D_HID), lambda i: (i, 0)),
        out_shape=jax.ShapeDtypeStruct((N_NODES, D_HID), jnp.bfloat16),
    )(x, w)


def _mid(p, b1, w2p):
    def body(p_ref, b_ref, w_ref, o_ref):
        psum = p_ref[0].astype(jnp.float32) + p_ref[1].astype(jnp.float32)
        h = jnp.maximum(psum + b_ref[...], 0.0)
        o_ref[...] = jnp.dot(h, w_ref[...], preferred_element_type=jnp.float32
                             ).astype(jnp.bfloat16)

    return pl.pallas_call(
        body,
        grid=(5,),
        in_specs=[
            pl.BlockSpec((N_SC, ROW_BLK, D_HID), lambda i: (0, i, 0)),
            pl.BlockSpec((1, D_HID), lambda i: (0, 0)),
            pl.BlockSpec((D_HID, D_PAD), lambda i: (0, 0)),
        ],
        out_specs=pl.BlockSpec((ROW_BLK, D_PAD), lambda i: (i, 0)),
        out_shape=jax.ShapeDtypeStruct((N_NODES, D_PAD), jnp.bfloat16),
    )(p, b1, w2p)


def _final(q, b2p):
    def body(q_ref, b_ref, o_ref):
        o_ref[...] = (q_ref[0].astype(jnp.float32)
                      + q_ref[1].astype(jnp.float32) + b_ref[...])

    return pl.pallas_call(
        body,
        grid=(5,),
        in_specs=[
            pl.BlockSpec((N_SC, ROW_BLK, D_PAD), lambda i: (0, i, 0)),
            pl.BlockSpec((1, D_PAD), lambda i: (0, 0)),
        ],
        out_specs=pl.BlockSpec((ROW_BLK, D_PAD), lambda i: (i, 0)),
        out_shape=jax.ShapeDtypeStruct((N_NODES, D_PAD), jnp.float32),
    )(q, b2p)


@jax.jit
def kernel(x, adj, W1, b1, W2, b2):
    src2d = adj[0].reshape(-1, CHUNK)
    dst2d = adj[1].reshape(-1, CHUNK)
    w2p = jnp.pad(W2, ((0, 0), (0, D_PAD - N_CLASS)))
    b2p = jnp.pad(b2, (0, D_PAD - N_CLASS)).reshape(1, D_PAD)
    b1r = b1.reshape(1, D_HID)

    support1 = _matmul1(x, W1)
    p1 = _edge_aggregate(support1, src2d, dst2d, D_HID, jnp.bfloat16)
    support2 = _mid(p1, b1r, w2p)
    q = _edge_aggregate(support2, src2d, dst2d, D_PAD, jnp.bfloat16)
    out = _final(q, b2p)
    return out[:, :N_CLASS]


# speedup vs baseline: 2.3924x; 1.0711x over previous
import functools
import jax
import jax.numpy as jnp
from jax import lax
from jax.experimental import pallas as pl
from jax.experimental.pallas import tpu as pltpu
from jax.experimental.pallas import tpu_sc as plsc

N_NODES = 10000
N_EDGES = 320000
D_IN = 128
D_HID = 128
N_CLASS = 40
D_PAD = 64

N_SC = 2
N_TILES = 16
ROWS_PER_TILE = N_NODES // N_TILES
ROW_BLK = 2000
CHUNK = 125
S0 = 100
S1 = 60
NBUF = 5


def _edge_aggregate(sup, src2d, dst2d, d, dtype):
    mesh = plsc.VectorSubcoreMesh(core_axis_name="c", subcore_axis_name="s",
                                  num_cores=N_SC, num_subcores=N_TILES)

    def body(sup_hbm, src_hbm, dst_hbm, zeros_hbm, out_hbm,
             sidx_v, didx_v, rows_v, acc_sh, sem_g, sem_s):
        c = lax.axis_index("c")
        s = lax.axis_index("s")
        nt = jnp.where(c == 0, S0, S1)

        @pl.when(c == 0)
        def _():
            start = s * S0
            pltpu.sync_copy(src_hbm.at[pl.ds(start, S0)], sidx_v)
            pltpu.sync_copy(dst_hbm.at[pl.ds(start, S0)], didx_v)

        @pl.when(c == 1)
        def _():
            start = N_TILES * S0 + s * S1
            pltpu.sync_copy(src_hbm.at[pl.ds(start, S1)],
                            sidx_v.at[pl.ds(0, S1)])
            pltpu.sync_copy(dst_hbm.at[pl.ds(start, S1)],
                            didx_v.at[pl.ds(0, S1)])

        pltpu.sync_copy(zeros_hbm,
                        acc_sh.at[pl.ds(s * ROWS_PER_TILE, ROWS_PER_TILE)])
        plsc.subcore_barrier()

        def fire_gather(t, b):
            pltpu.async_copy(sup_hbm.at[sidx_v.at[t]], rows_v.at[b],
                             sem_g.at[b])

        def wait_gather(t, b):
            pltpu.make_async_copy(sup_hbm.at[sidx_v.at[t]], rows_v.at[b],
                                  sem_g.at[b]).wait()

        def fire_scatter(t, b):
            pltpu.async_copy(rows_v.at[b], acc_sh.at[didx_v.at[t]],
                             sem_s.at[b], add=True)

        def wait_scatter(t, b):
            pltpu.make_async_copy(rows_v.at[b], acc_sh.at[didx_v.at[t]],
                                  sem_s.at[b]).wait()

        for b in range(NBUF - 1):
            fire_gather(b, b)

        def slot_step(t, sb, first=False, last_grp=False):
            sbp = (sb + NBUF - 1) % NBUF
            wait_gather(t, sb)
            fire_scatter(t, sb)
            if not first:
                wait_scatter(t - 1, sbp)
            if not last_grp:
                fire_gather(t + NBUF - 1, sbp)

        for sb in range(NBUF):
            slot_step(sb, sb, first=(sb == 0))

        def group(g, carry):
            t0 = g * NBUF
            for sb in range(NBUF):
                slot_step(t0 + sb, sb)
            return carry

        lax.fori_loop(1, nt // NBUF - 1, group, 0)

        t0 = nt - NBUF
        for sb in range(NBUF):
            slot_step(t0 + sb, sb, last_grp=(sb != 0))
        wait_scatter(nt - 1, NBUF - 1)

        plsc.subcore_barrier()
        pltpu.sync_copy(acc_sh.at[pl.ds(s * ROWS_PER_TILE, ROWS_PER_TILE)],
                        out_hbm.at[c, pl.ds(s * ROWS_PER_TILE,
                                            ROWS_PER_TILE)])

    kern = pl.kernel(
        body,
        out_type=jax.ShapeDtypeStruct((N_SC, N_NODES, d), dtype),
        mesh=mesh,
        scratch_types=[
            pltpu.VMEM((S0, CHUNK), jnp.int32),
            pltpu.VMEM((S0, CHUNK), jnp.int32),
            pltpu.VMEM((NBUF, CHUNK, d), dtype),
            pltpu.VMEM_SHARED((N_NODES, d), dtype),
            pltpu.SemaphoreType.DMA((NBUF,)),
            pltpu.SemaphoreType.DMA((NBUF,)),
        ],
        compiler_params=pltpu.CompilerParams(use_tc_tiling_on_sc=False),
    )
    zeros = jnp.zeros((ROWS_PER_TILE, d), dtype)
    return kern(sup, src2d, dst2d, zeros)


def _matmul1(x, w):
    def body(x_ref, w_ref, o_ref):
        o_ref[...] = jnp.dot(x_ref[...], w_ref[...],
                             preferred_element_type=jnp.float32
                             ).astype(jnp.bfloat16)

    return pl.pallas_call(
        body,
        grid=(5,),
        in_specs=[
            pl.BlockSpec((ROW_BLK, D_IN), lambda i: (i, 0)),
            pl.BlockSpec((D_IN, D_HID), lambda i: (0, 0)),
        ],
        out_specs=pl.BlockSpec((ROW_BLK, D_HID), lambda i: (i, 0)),
        out_shape=jax.ShapeDtypeStruct((N_NODES, D_HID), jnp.bfloat16),
    )(x, w)


def _mid(p, b1, w2p):
    def body(p_ref, b_ref, w_ref, o_ref):
        psum = p_ref[0].astype(jnp.float32) + p_ref[1].astype(jnp.float32)
        h = jnp.maximum(psum + b_ref[...], 0.0)
        o_ref[...] = jnp.dot(h, w_ref[...], preferred_element_type=jnp.float32
                             ).astype(jnp.bfloat16)

    return pl.pallas_call(
        body,
        grid=(5,),
        in_specs=[
            pl.BlockSpec((N_SC, ROW_BLK, D_HID), lambda i: (0, i, 0)),
            pl.BlockSpec((1, D_HID), lambda i: (0, 0)),
            pl.BlockSpec((D_HID, D_PAD), lambda i: (0, 0)),
        ],
        out_specs=pl.BlockSpec((ROW_BLK, D_PAD), lambda i: (i, 0)),
        out_shape=jax.ShapeDtypeStruct((N_NODES, D_PAD), jnp.bfloat16),
    )(p, b1, w2p)


def _final(q, b2p):
    def body(q_ref, b_ref, o_ref):
        full = (q_ref[0].astype(jnp.float32)
                + q_ref[1].astype(jnp.float32) + b_ref[...])
        o_ref[...] = full[:, :N_CLASS]

    return pl.pallas_call(
        body,
        grid=(5,),
        in_specs=[
            pl.BlockSpec((N_SC, ROW_BLK, D_PAD), lambda i: (0, i, 0)),
            pl.BlockSpec((1, D_PAD), lambda i: (0, 0)),
        ],
        out_specs=pl.BlockSpec((ROW_BLK, N_CLASS), lambda i: (i, 0)),
        out_shape=jax.ShapeDtypeStruct((N_NODES, N_CLASS), jnp.float32),
    )(q, b2p)


@jax.jit
def kernel(x, adj, W1, b1, W2, b2):
    src2d = adj[0].reshape(-1, CHUNK)
    dst2d = adj[1].reshape(-1, CHUNK)
    w2p = jnp.pad(W2, ((0, 0), (0, D_PAD - N_CLASS)))
    b2p = jnp.pad(b2, (0, D_PAD - N_CLASS)).reshape(1, D_PAD)
    b1r = b1.reshape(1, D_HID)

    support1 = _matmul1(x, W1)
    p1 = _edge_aggregate(support1, src2d, dst2d, D_HID, jnp.bfloat16)
    support2 = _mid(p1, b1r, w2p)
    q = _edge_aggregate(support2, src2d, dst2d, D_PAD, jnp.bfloat16)
    return _final(q, b2p)


# speedup vs baseline: 2.4996x; 1.0448x over previous
import functools
import jax
import jax.numpy as jnp
from jax import lax
from jax.experimental import pallas as pl
from jax.experimental.pallas import tpu as pltpu
from jax.experimental.pallas import tpu_sc as plsc

N_NODES = 10000
N_EDGES = 320000
D_IN = 128
D_HID = 128
N_CLASS = 40
D_PAD = 64

N_SC = 2
N_TILES = 16
ROWS_PER_TILE = N_NODES // N_TILES
ROW_BLK = 2000
CHUNK = 125
S0 = 85
S1 = 75
NBUF = 5


def _edge_aggregate(sup, src2d, dst2d, d, dtype):
    mesh = plsc.VectorSubcoreMesh(core_axis_name="c", subcore_axis_name="s",
                                  num_cores=N_SC, num_subcores=N_TILES)

    def body(sup_hbm, src_hbm, dst_hbm, zeros_hbm, out_hbm,
             sidx_v, didx_v, rows_v, acc_sh, sem_g, sem_s):
        c = lax.axis_index("c")
        s = lax.axis_index("s")
        nt = jnp.where(c == 0, S0, S1)

        @pl.when(c == 0)
        def _():
            start = s * S0
            pltpu.sync_copy(src_hbm.at[pl.ds(start, S0)], sidx_v)
            pltpu.sync_copy(dst_hbm.at[pl.ds(start, S0)], didx_v)

        @pl.when(c == 1)
        def _():
            start = N_TILES * S0 + s * S1
            pltpu.sync_copy(src_hbm.at[pl.ds(start, S1)],
                            sidx_v.at[pl.ds(0, S1)])
            pltpu.sync_copy(dst_hbm.at[pl.ds(start, S1)],
                            didx_v.at[pl.ds(0, S1)])

        pltpu.sync_copy(zeros_hbm,
                        acc_sh.at[pl.ds(s * ROWS_PER_TILE, ROWS_PER_TILE)])
        plsc.subcore_barrier()

        def fire_gather(t, b):
            pltpu.async_copy(sup_hbm.at[sidx_v.at[t]], rows_v.at[b],
                             sem_g.at[b])

        def wait_gather(t, b):
            pltpu.make_async_copy(sup_hbm.at[sidx_v.at[t]], rows_v.at[b],
                                  sem_g.at[b]).wait()

        def fire_scatter(t, b):
            pltpu.async_copy(rows_v.at[b], acc_sh.at[didx_v.at[t]],
                             sem_s.at[b], add=True)

        def wait_scatter(t, b):
            pltpu.make_async_copy(rows_v.at[b], acc_sh.at[didx_v.at[t]],
                                  sem_s.at[b]).wait()

        for b in range(NBUF - 1):
            fire_gather(b, b)

        def slot_step(t, sb, first=False, last_grp=False):
            sbp = (sb + NBUF - 1) % NBUF
            wait_gather(t, sb)
            fire_scatter(t, sb)
            if not first:
                wait_scatter(t - 1, sbp)
            if not last_grp:
                fire_gather(t + NBUF - 1, sbp)

        for sb in range(NBUF):
            slot_step(sb, sb, first=(sb == 0))

        def group(g, carry):
            t0 = g * NBUF
            for sb in range(NBUF):
                slot_step(t0 + sb, sb)
            return carry

        lax.fori_loop(1, nt // NBUF - 1, group, 0)

        t0 = nt - NBUF
        for sb in range(NBUF):
            slot_step(t0 + sb, sb, last_grp=(sb != 0))
        wait_scatter(nt - 1, NBUF - 1)

        plsc.subcore_barrier()
        pltpu.sync_copy(acc_sh.at[pl.ds(s * ROWS_PER_TILE, ROWS_PER_TILE)],
                        out_hbm.at[c, pl.ds(s * ROWS_PER_TILE,
                                            ROWS_PER_TILE)])

    kern = pl.kernel(
        body,
        out_type=jax.ShapeDtypeStruct((N_SC, N_NODES, d), dtype),
        mesh=mesh,
        scratch_types=[
            pltpu.VMEM((S0, CHUNK), jnp.int32),
            pltpu.VMEM((S0, CHUNK), jnp.int32),
            pltpu.VMEM((NBUF, CHUNK, d), dtype),
            pltpu.VMEM_SHARED((N_NODES, d), dtype),
            pltpu.SemaphoreType.DMA((NBUF,)),
            pltpu.SemaphoreType.DMA((NBUF,)),
        ],
        compiler_params=pltpu.CompilerParams(use_tc_tiling_on_sc=False),
    )
    zeros = jnp.zeros((ROWS_PER_TILE, d), dtype)
    return kern(sup, src2d, dst2d, zeros)


def _matmul1(x, w):
    def body(x_ref, w_ref, o_ref):
        o_ref[...] = jnp.dot(x_ref[...], w_ref[...],
                             preferred_element_type=jnp.float32
                             ).astype(jnp.bfloat16)

    return pl.pallas_call(
        body,
        grid=(5,),
        in_specs=[
            pl.BlockSpec((ROW_BLK, D_IN), lambda i: (i, 0)),
            pl.BlockSpec((D_IN, D_HID), lambda i: (0, 0)),
        ],
        out_specs=pl.BlockSpec((ROW_BLK, D_HID), lambda i: (i, 0)),
        out_shape=jax.ShapeDtypeStruct((N_NODES, D_HID), jnp.bfloat16),
    )(x, w)


def _mid(p, b1, w2p):
    def body(p_ref, b_ref, w_ref, o_ref):
        psum = p_ref[0].astype(jnp.float32) + p_ref[1].astype(jnp.float32)
        h = jnp.maximum(psum + b_ref[...], 0.0)
        o_ref[...] = jnp.dot(h, w_ref[...], preferred_element_type=jnp.float32
                             ).astype(jnp.bfloat16)

    return pl.pallas_call(
        body,
        grid=(5,),
        in_specs=[
            pl.BlockSpec((N_SC, ROW_BLK, D_HID), lambda i: (0, i, 0)),
            pl.BlockSpec((1, D_HID), lambda i: (0, 0)),
            pl.BlockSpec((D_HID, D_PAD), lambda i: (0, 0)),
        ],
        out_specs=pl.BlockSpec((ROW_BLK, D_PAD), lambda i: (i, 0)),
        out_shape=jax.ShapeDtypeStruct((N_NODES, D_PAD), jnp.bfloat16),
    )(p, b1, w2p)


def _final(q, b2p):
    def body(q_ref, b_ref, o_ref):
        full = (q_ref[0].astype(jnp.float32)
                + q_ref[1].astype(jnp.float32) + b_ref[...])
        o_ref[...] = full[:, :N_CLASS]

    return pl.pallas_call(
        body,
        grid=(5,),
        in_specs=[
            pl.BlockSpec((N_SC, ROW_BLK, D_PAD), lambda i: (0, i, 0)),
            pl.BlockSpec((1, D_PAD), lambda i: (0, 0)),
        ],
        out_specs=pl.BlockSpec((ROW_BLK, N_CLASS), lambda i: (i, 0)),
        out_shape=jax.ShapeDtypeStruct((N_NODES, N_CLASS), jnp.float32),
    )(q, b2p)


@jax.jit
def kernel(x, adj, W1, b1, W2, b2):
    src2d = adj[0].reshape(-1, CHUNK)
    dst2d = adj[1].reshape(-1, CHUNK)
    w2p = jnp.pad(W2, ((0, 0), (0, D_PAD - N_CLASS)))
    b2p = jnp.pad(b2, (0, D_PAD - N_CLASS)).reshape(1, D_PAD)
    b1r = b1.reshape(1, D_HID)

    support1 = _matmul1(x, W1)
    p1 = _edge_aggregate(support1, src2d, dst2d, D_HID, jnp.bfloat16)
    support2 = _mid(p1, b1r, w2p)
    q = _edge_aggregate(support2, src2d, dst2d, D_PAD, jnp.bfloat16)
    return _final(q, b2p)
